# baseline (speedup 1.0000x reference)
import functools

import jax
import jax.numpy as jnp
from jax import lax
from jax.experimental import pallas as pl
from jax.experimental.pallas import tpu as pltpu
from jax.experimental.pallas import tpu_sc as plsc

B = 16384
D = 64
SR = 2 * D
NC = 2
NS = 16
L = 16
NW = NC * NS
BPW = B // NW
CH = 128
NCH = BPW // CH

_mesh = plsc.VectorSubcoreMesh(core_axis_name="c", subcore_axis_name="s")

_params = pltpu.CompilerParams(
    use_tc_tiling_on_sc=True,
    needs_layout_passes=False,
)


@functools.partial(
    pl.kernel,
    mesh=_mesh,
    out_type=jax.ShapeDtypeStruct((B,), jnp.float32),
    scratch_types=[
        pltpu.VMEM((4 * BPW,), jnp.int32),
        pltpu.VMEM((BPW // 2, SR), jnp.float32),
        pltpu.VMEM((BPW // 2, SR), jnp.float32),
        pltpu.VMEM((BPW,), jnp.float32),
        pltpu.SemaphoreType.DMA,
    ],
    compiler_params=_params,
)
def _sc_dot(ids_hbm, ut_hbm, bt_hbm, out_hbm, idxv, urows, brows, outv, sem):
    wid = lax.axis_index("s") * NC + lax.axis_index("c")

    for p in range(4):
        pltpu.sync_copy(ids_hbm.at[pl.ds(p * B + wid * BPW, BPW)],
                        idxv.at[pl.ds(p * BPW, BPW)])

    lane = lax.broadcasted_iota(jnp.int32, (L,), 0)
    dn = lax.GatherDimensionNumbers(
        offset_dims=(), collapsed_slice_dims=(0,), start_index_map=(0,))
    HB = BPW // 2

    for h in range(2):
        hb = h * HB
        for j in range(HB // CH):
            pltpu.async_copy(
                ut_hbm.at[idxv.at[pl.ds(hb + j * CH, CH)]],
                urows.at[pl.ds(j * CH, CH)], sem)
            pltpu.async_copy(
                bt_hbm.at[idxv.at[pl.ds(BPW + hb + j * CH, CH)]],
                brows.at[pl.ds(j * CH, CH)], sem)
        pltpu.make_async_copy(ut_hbm.at[pl.ds(0, HB)], urows, sem).wait()
        pltpu.make_async_copy(bt_hbm.at[pl.ds(0, HB)], brows, sem).wait()

        def group(g, carry):
            r0 = g * L
            pu = idxv[pl.ds(2 * BPW + hb + r0, L)]
            pb = idxv[pl.ds(3 * BPW + hb + r0, L)]
            acc = jnp.zeros((L,), jnp.float32)
            for j in range(L):
                r = r0 + j
                jidx = jnp.full((L,), j, jnp.int32)
                puj = lax.gather(pu, jidx[:, None], dn, (1,),
                                 mode=lax.GatherScatterMode.PROMISE_IN_BOUNDS)
                pbj = lax.gather(pb, jidx[:, None], dn, (1,),
                                 mode=lax.GatherScatterMode.PROMISE_IN_BOUNDS)
                s = jnp.zeros((L,), jnp.float32)
                for k in range(D // L):
                    ulo = urows[r, pl.ds(k * L, L)]
                    uhi = urows[r, pl.ds(D + k * L, L)]
                    vlo = brows[r, pl.ds(k * L, L)]
                    vhi = brows[r, pl.ds(D + k * L, L)]
                    u = jnp.where(puj > 0, uhi, ulo)
                    v = jnp.where(pbj > 0, vhi, vlo)
                    s = s + u * v
                acc = jnp.where(lane == j, jnp.sum(s), acc)
            outv[pl.ds(hb + r0, L)] = acc
            return carry

        lax.fori_loop(0, HB // L, group, 0, unroll=False)

    pltpu.sync_copy(outv, out_hbm.at[pl.ds(wid * BPW, BPW)])


@jax.jit
def kernel(user_ids, book_ids, user_table, book_table):
    uid = user_ids.reshape(B)
    bid = book_ids.reshape(B)
    ids = jnp.concatenate([uid >> 1, bid >> 1, uid & 1, bid & 1])
    ut2 = user_table.reshape(-1, SR)
    bt2 = book_table.reshape(-1, SR)
    out = _sc_dot(ids, ut2, bt2)
    return out.reshape(B, 1)


# speedup vs baseline: 2.3449x; 2.3449x over previous
import functools

import jax
import jax.numpy as jnp
from jax import lax
from jax.experimental import pallas as pl
from jax.experimental.pallas import tpu as pltpu
from jax.experimental.pallas import tpu_sc as plsc

B = 16384
D = 64
SR = 2 * D
NC = 2
NS = 16
L = 16
NW = NC * NS
BPW = B // NW
CH = 128
USERS = 1000000
UPAD = 1000064
CWID = 640
CMAXSTART = UPAD - CWID
SLOTS = 544
IDPAD = NW * BPW + (SLOTS - BPW) + L
TRASH = B

_mesh = plsc.VectorSubcoreMesh(core_axis_name="c", subcore_axis_name="s")
_params = pltpu.CompilerParams(use_tc_tiling_on_sc=True, needs_layout_passes=False)


@functools.partial(
    pl.kernel,
    mesh=_mesh,
    out_type=jax.ShapeDtypeStruct((B + 8, 128), jnp.float32),
    scratch_types=[
        pltpu.VMEM((64, CWID), jnp.float32),
        pltpu.VMEM((SLOTS + L,), jnp.int32),
        pltpu.VMEM((4, CH), jnp.int32),
        pltpu.VMEM((SLOTS, 128), jnp.float32),
        pltpu.SemaphoreType.DMA,
    ],
    compiler_params=_params,
)
def _k1_extract(uids_hbm, pos_hbm, ut_hbm, emb_hbm, chunk, uidv, psc, embv, sem):
    wid = lax.axis_index("s") * NC + lax.axis_index("c")
    base = wid * BPW
    pltpu.sync_copy(uids_hbm.at[pl.ds(base, SLOTS + L)], uidv)
    for j in range(4):
        pltpu.sync_copy(pos_hbm.at[pl.ds(base + j * CH, CH)], psc.at[j])

    d16 = lax.broadcasted_iota(jnp.int32, (L,), 0)

    def body(carry):
        r = carry
        uv0 = uidv[pl.ds(r, L)]
        u0 = uv0[0]
        cstart = jnp.minimum(u0 & ~jnp.int32(511), jnp.int32(CMAXSTART))
        pltpu.sync_copy(
            ut_hbm.at[:, pl.ds(pl.multiple_of(cstart, 128), CWID)], chunk)
        cend = cstart + CWID
        m = (uv0 < cend).astype(jnp.int32)
        n = jnp.sum(m)
        for j in range(L):
            uj = uv0[j]
            cc = jnp.minimum(jnp.maximum(uj - cstart, 0), CWID - 1)
            ccv = jnp.zeros((L,), jnp.int32) + cc
            for k in range(4):
                g = plsc.load_gather(chunk, [d16 + k * L, ccv])
                embv[r + j, pl.ds(k * L, L)] = g
        return r + n

    lax.while_loop(lambda r: r < BPW, body, jnp.int32(0))

    for j in range(4):
        pltpu.async_copy(embv.at[pl.ds(j * CH, CH)],
                         emb_hbm.at[psc.at[j]], sem)
    for j in range(4):
        pltpu.make_async_copy(emb_hbm.at[pl.ds(0, CH)],
                              embv.at[pl.ds(0, CH)], sem).wait()


@functools.partial(
    pl.kernel,
    mesh=_mesh,
    out_type=jax.ShapeDtypeStruct((B,), jnp.float32),
    scratch_types=[
        pltpu.VMEM((2 * BPW,), jnp.int32),
        pltpu.VMEM((BPW // 2, 128), jnp.float32),
        pltpu.VMEM((BPW // 2, SR), jnp.float32),
        pltpu.VMEM((BPW,), jnp.float32),
        pltpu.SemaphoreType.DMA,
    ],
    compiler_params=_params,
)
def _k2_dot(ids_hbm, emb_hbm, bt_hbm, out_hbm, idxv, urows, brows, outv, sem):
    wid = lax.axis_index("s") * NC + lax.axis_index("c")
    for p in range(2):
        pltpu.sync_copy(ids_hbm.at[pl.ds(p * B + wid * BPW, BPW)],
                        idxv.at[pl.ds(p * BPW, BPW)])

    lane = lax.broadcasted_iota(jnp.int32, (L,), 0)
    dn = lax.GatherDimensionNumbers(
        offset_dims=(), collapsed_slice_dims=(0,), start_index_map=(0,))
    HB = BPW // 2

    for h in range(2):
        hb = h * HB
        pltpu.sync_copy(emb_hbm.at[pl.ds(wid * BPW + hb, HB), :], urows)
        for j in range(HB // CH):
            pltpu.async_copy(
                bt_hbm.at[idxv.at[pl.ds(hb + j * CH, CH)]],
                brows.at[pl.ds(j * CH, CH)], sem)
        pltpu.make_async_copy(bt_hbm.at[pl.ds(0, HB)], brows, sem).wait()

        def group(g, carry):
            r0 = g * L
            pb = idxv[pl.ds(BPW + hb + r0, L)]
            acc = jnp.zeros((L,), jnp.float32)
            for j in range(L):
                r = r0 + j
                jidx = jnp.full((L,), j, jnp.int32)
                pbj = lax.gather(pb, jidx[:, None], dn, (1,),
                                 mode=lax.GatherScatterMode.PROMISE_IN_BOUNDS)
                s = jnp.zeros((L,), jnp.float32)
                for k in range(D // L):
                    u = urows[r, pl.ds(k * L, L)]
                    vlo = brows[r, pl.ds(k * L, L)]
                    vhi = brows[r, pl.ds(D + k * L, L)]
                    v = jnp.where(pbj > 0, vhi, vlo)
                    s = s + u * v
                acc = jnp.where(lane == j, jnp.sum(s), acc)
            outv[pl.ds(hb + r0, L)] = acc
            return carry

        lax.fori_loop(0, HB // L, group, 0, unroll=False)

    pltpu.sync_copy(outv, out_hbm.at[pl.ds(wid * BPW, BPW)])


@jax.jit
def kernel(user_ids, book_ids, user_table, book_table):
    uid = user_ids.reshape(B)
    bid = book_ids.reshape(B)
    perm = jnp.argsort(uid)
    uid_sorted = uid[perm].astype(jnp.int32)
    npad = IDPAD - B
    uids_pad = jnp.concatenate(
        [uid_sorted, jnp.full((npad,), jnp.int32(0x7FFFFFFF))])
    pos_pad = jnp.concatenate(
        [perm.astype(jnp.int32), jnp.full((npad,), jnp.int32(TRASH))])
    emb = _k1_extract(uids_pad, pos_pad, user_table.T)
    ids_book = jnp.concatenate([bid >> 1, bid & 1])
    bt2 = book_table.reshape(-1, SR)
    out = _k2_dot(ids_book, emb, bt2)
    return out.reshape(B, 1)


# speedup vs baseline: 2.4754x; 1.0557x over previous
import functools

import jax
import jax.numpy as jnp
from jax import lax
from jax.experimental import pallas as pl
from jax.experimental.pallas import tpu as pltpu
from jax.experimental.pallas import tpu_sc as plsc

B = 16384
D = 64
NC = 2
NS = 16
L = 16
NW = NC * NS
BPW = B // NW
CH = 128
CWID = 384
SLOTS = 544
IDPAD = B + (SLOTS - BPW) + L
TRASH = B

_mesh = plsc.VectorSubcoreMesh(core_axis_name="c", subcore_axis_name="s")
_params = pltpu.CompilerParams(use_tc_tiling_on_sc=True, needs_layout_passes=False)


def _make_extract(cmaxstart):
    @functools.partial(
        pl.kernel,
        mesh=_mesh,
        out_type=jax.ShapeDtypeStruct((B + 8, 128), jnp.float32),
        scratch_types=[
            pltpu.VMEM((2 * 64, CWID), jnp.float32),
            pltpu.VMEM((SLOTS + L,), jnp.int32),
            pltpu.VMEM((4, CH), jnp.int32),
            pltpu.VMEM((SLOTS, 128), jnp.float32),
            pltpu.SemaphoreType.DMA,
            pltpu.SemaphoreType.DMA,
            pltpu.SemaphoreType.DMA,
        ],
        compiler_params=_params,
    )
    def _extract(ids_hbm, pos_hbm, tab_hbm, emb_hbm,
                 chunk, idv, psc, embv, sem, csem0, csem1):
        wid = lax.axis_index("s") * NC + lax.axis_index("c")
        base = wid * BPW
        pltpu.sync_copy(ids_hbm.at[pl.ds(base, SLOTS + L)], idv)
        for j in range(4):
            pltpu.sync_copy(pos_hbm.at[pl.ds(base + j * CH, CH)], psc.at[j])

        d16 = lax.broadcasted_iota(jnp.int32, (L,), 0)

        def chunk_start(r):
            u0 = idv[pl.ds(r, L)][0]
            return jnp.minimum(u0 & ~jnp.int32(255), jnp.int32(cmaxstart))

        def fire(q, cstart):
            csem = csem0 if q == 0 else csem1
            pltpu.async_copy(
                tab_hbm.at[:, pl.ds(pl.multiple_of(cstart, 128), CWID)],
                chunk.at[pl.ds(q * 64, 64)], csem)

        def drain(q):
            csem = csem0 if q == 0 else csem1
            pltpu.make_async_copy(
                tab_hbm.at[:, pl.ds(0, CWID)],
                chunk.at[pl.ds(q * 64, 64)], csem).wait()

        cs0 = chunk_start(jnp.int32(0))
        fire(0, cs0)

        def body(carry):
            r, p, cstart = carry
            uv0 = idv[pl.ds(r, L)]
            m = (uv0 < cstart + CWID).astype(jnp.int32)
            n = jnp.sum(m)
            csn = chunk_start(r + n)

            @pl.when(p == 0)
            def _():
                fire(1, csn)
                drain(0)

            @pl.when(p == 1)
            def _():
                fire(0, csn)
                drain(1)

            rowbase = p * 64
            for j in range(L):
                uj = uv0[j]
                cc = jnp.minimum(jnp.maximum(uj - cstart, 0), CWID - 1)
                ccv = jnp.zeros((L,), jnp.int32) + cc
                for k in range(4):
                    g = plsc.load_gather(chunk, [rowbase + d16 + k * L, ccv])
                    embv[r + j, pl.ds(k * L, L)] = g
            return r + n, 1 - p, csn

        rf, pf, _ = lax.while_loop(
            lambda c: c[0] < BPW, body, (jnp.int32(0), jnp.int32(0), cs0))

        @pl.when(pf == 0)
        def _():
            drain(0)

        @pl.when(pf == 1)
        def _():
            drain(1)

        for j in range(4):
            pltpu.async_copy(embv.at[pl.ds(j * CH, CH)],
                             emb_hbm.at[psc.at[j]], sem)
        for j in range(4):
            pltpu.make_async_copy(emb_hbm.at[pl.ds(0, CH)],
                                  embv.at[pl.ds(0, CH)], sem).wait()

    return _extract


_extract_user = _make_extract(1000064 - CWID)
_extract_book = _make_extract(100096 - CWID)


@functools.partial(
    pl.kernel,
    mesh=_mesh,
    out_type=jax.ShapeDtypeStruct((B,), jnp.float32),
    scratch_types=[
        pltpu.VMEM((BPW // 2, 128), jnp.float32),
        pltpu.VMEM((BPW // 2, 128), jnp.float32),
        pltpu.VMEM((BPW,), jnp.float32),
    ],
    compiler_params=_params,
)
def _dot(uemb_hbm, bemb_hbm, out_hbm, urows, brows, outv):
    wid = lax.axis_index("s") * NC + lax.axis_index("c")
    lane = lax.broadcasted_iota(jnp.int32, (L,), 0)
    HB = BPW // 2

    for h in range(2):
        hb = h * HB
        pltpu.sync_copy(uemb_hbm.at[pl.ds(wid * BPW + hb, HB), :], urows)
        pltpu.sync_copy(bemb_hbm.at[pl.ds(wid * BPW + hb, HB), :], brows)

        def group(g, carry):
            r0 = g * L
            acc = jnp.zeros((L,), jnp.float32)
            for j in range(L):
                r = r0 + j
                s = jnp.zeros((L,), jnp.float32)
                for k in range(D // L):
                    s = s + (urows[r, pl.ds(k * L, L)]
                             * brows[r, pl.ds(k * L, L)])
                acc = jnp.where(lane == j, jnp.sum(s), acc)
            outv[pl.ds(hb + r0, L)] = acc
            return carry

        lax.fori_loop(0, HB // L, group, 0, unroll=False)

    pltpu.sync_copy(outv, out_hbm.at[pl.ds(wid * BPW, BPW)])


def _sorted_ids(ids):
    perm = jnp.argsort(ids)
    npad = IDPAD - B
    ids_pad = jnp.concatenate(
        [ids[perm].astype(jnp.int32),
         jnp.full((npad,), jnp.int32(0x7FFFFFFF))])
    pos_pad = jnp.concatenate(
        [perm.astype(jnp.int32), jnp.full((npad,), jnp.int32(TRASH))])
    return ids_pad, pos_pad


@jax.jit
def kernel(user_ids, book_ids, user_table, book_table):
    uid = user_ids.reshape(B)
    bid = book_ids.reshape(B)
    up, upos = _sorted_ids(uid)
    bp, bpos = _sorted_ids(bid)
    uemb = _extract_user(up, upos, user_table.T)
    bemb = _extract_book(bp, bpos, book_table.T)
    out = _dot(uemb, bemb)
    return out.reshape(B, 1)


# speedup vs baseline: 2.8836x; 1.1649x over previous
import functools

import jax
import jax.numpy as jnp
from jax import lax
from jax.experimental import pallas as pl
from jax.experimental.pallas import tpu as pltpu
from jax.experimental.pallas import tpu_sc as plsc

B = 16384
D = 64
NC = 2
NS = 16
L = 16
NW = NC * NS
BPW = B // NW
CH = 128
CWID = 384
SLOTS = 544
IDPAD = B + (SLOTS - BPW) + L
TRASH = B

_mesh = plsc.VectorSubcoreMesh(core_axis_name="c", subcore_axis_name="s")
_params = pltpu.CompilerParams(use_tc_tiling_on_sc=True, needs_layout_passes=False)


def _make_extract(cmaxstart):
    @functools.partial(
        pl.kernel,
        mesh=_mesh,
        out_type=jax.ShapeDtypeStruct((B + 8, 128), jnp.float32),
        scratch_types=[
            pltpu.VMEM((2 * 64, CWID), jnp.float32),
            pltpu.VMEM((SLOTS + L,), jnp.int32),
            pltpu.VMEM((4, CH), jnp.int32),
            pltpu.VMEM((SLOTS, 128), jnp.float32),
            pltpu.SemaphoreType.DMA,
            pltpu.SemaphoreType.DMA,
            pltpu.SemaphoreType.DMA,
        ],
        compiler_params=_params,
    )
    def _extract(ids_hbm, pos_hbm, tab_hbm, emb_hbm,
                 chunk, idv, psc, embv, sem, csem0, csem1):
        wid = lax.axis_index("s") * NC + lax.axis_index("c")
        base = wid * BPW
        pltpu.sync_copy(ids_hbm.at[pl.ds(base, SLOTS + L)], idv)
        for j in range(4):
            pltpu.sync_copy(pos_hbm.at[pl.ds(base + j * CH, CH)], psc.at[j])

        d16 = lax.broadcasted_iota(jnp.int32, (L,), 0)

        def chunk_start(r):
            u0 = idv[pl.ds(r, L)][0]
            return jnp.minimum((u0 // CWID) * CWID, jnp.int32(cmaxstart))

        def fire(q, cstart):
            csem = csem0 if q == 0 else csem1
            pltpu.async_copy(
                tab_hbm.at[:, pl.ds(pl.multiple_of(cstart, 128), CWID)],
                chunk.at[pl.ds(q * 64, 64)], csem)

        def drain(q):
            csem = csem0 if q == 0 else csem1
            pltpu.make_async_copy(
                tab_hbm.at[:, pl.ds(0, CWID)],
                chunk.at[pl.ds(q * 64, 64)], csem).wait()

        cs0 = chunk_start(jnp.int32(0))
        fire(0, cs0)

        def body(carry):
            r, p, cstart, fresh = carry
            uv0 = idv[pl.ds(r, L)]
            m = (uv0 < cstart + CWID).astype(jnp.int32)
            n = jnp.sum(m)
            csn = chunk_start(r + n)
            moved = csn != cstart

            @pl.when(moved & (p == 0))
            def _():
                fire(1, csn)

            @pl.when(moved & (p == 1))
            def _():
                fire(0, csn)

            @pl.when((fresh == 1) & (p == 0))
            def _():
                drain(0)

            @pl.when((fresh == 1) & (p == 1))
            def _():
                drain(1)

            rowbase = p * 64
            for j in range(L):
                uj = uv0[j]
                cc = jnp.minimum(jnp.maximum(uj - cstart, 0), CWID - 1)
                ccv = jnp.zeros((L,), jnp.int32) + cc
                for k in range(4):
                    g = plsc.load_gather(chunk, [rowbase + d16 + k * L, ccv])
                    embv[r + j, pl.ds(k * L, L)] = g
            pn = jnp.where(moved, 1 - p, p)
            return r + n, pn, csn, moved.astype(jnp.int32)

        rf, pf, _, ff = lax.while_loop(
            lambda c: c[0] < BPW, body,
            (jnp.int32(0), jnp.int32(0), cs0, jnp.int32(1)))

        @pl.when((ff == 1) & (pf == 0))
        def _():
            drain(0)

        @pl.when((ff == 1) & (pf == 1))
        def _():
            drain(1)

        for j in range(4):
            pltpu.async_copy(embv.at[pl.ds(j * CH, CH)],
                             emb_hbm.at[psc.at[j]], sem)
        for j in range(4):
            pltpu.make_async_copy(emb_hbm.at[pl.ds(0, CH)],
                                  embv.at[pl.ds(0, CH)], sem).wait()

    return _extract


_extract_user = _make_extract(1000064 - CWID)
_extract_book = _make_extract(100096 - CWID)


@functools.partial(
    pl.kernel,
    mesh=_mesh,
    out_type=jax.ShapeDtypeStruct((B,), jnp.float32),
    scratch_types=[
        pltpu.VMEM((BPW // 2, 128), jnp.float32),
        pltpu.VMEM((BPW // 2, 128), jnp.float32),
        pltpu.VMEM((BPW,), jnp.float32),
    ],
    compiler_params=_params,
)
def _dot(uemb_hbm, bemb_hbm, out_hbm, urows, brows, outv):
    wid = lax.axis_index("s") * NC + lax.axis_index("c")
    lane = lax.broadcasted_iota(jnp.int32, (L,), 0)
    HB = BPW // 2

    for h in range(2):
        hb = h * HB
        pltpu.sync_copy(uemb_hbm.at[pl.ds(wid * BPW + hb, HB), :], urows)
        pltpu.sync_copy(bemb_hbm.at[pl.ds(wid * BPW + hb, HB), :], brows)

        def group(g, carry):
            r0 = g * L
            acc = jnp.zeros((L,), jnp.float32)
            for j in range(L):
                r = r0 + j
                s = jnp.zeros((L,), jnp.float32)
                for k in range(D // L):
                    s = s + (urows[r, pl.ds(k * L, L)]
                             * brows[r, pl.ds(k * L, L)])
                acc = jnp.where(lane == j, jnp.sum(s), acc)
            outv[pl.ds(hb + r0, L)] = acc
            return carry

        lax.fori_loop(0, HB // L, group, 0, unroll=False)

    pltpu.sync_copy(outv, out_hbm.at[pl.ds(wid * BPW, BPW)])


def _sorted_ids(ids):
    perm = jnp.argsort(ids)
    npad = IDPAD - B
    ids_pad = jnp.concatenate(
        [ids[perm].astype(jnp.int32),
         jnp.full((npad,), jnp.int32(0x7FFFFFFF))])
    pos_pad = jnp.concatenate(
        [perm.astype(jnp.int32), jnp.full((npad,), jnp.int32(TRASH))])
    return ids_pad, pos_pad


@jax.jit
def kernel(user_ids, book_ids, user_table, book_table):
    uid = user_ids.reshape(B)
    bid = book_ids.reshape(B)
    up, upos = _sorted_ids(uid)
    bp, bpos = _sorted_ids(bid)
    uemb = _extract_user(up, upos, user_table.T)
    bemb = _extract_book(bp, bpos, book_table.T)
    out = _dot(uemb, bemb)
    return out.reshape(B, 1)


# speedup vs baseline: 3.1392x; 1.0886x over previous
import functools

import jax
import jax.numpy as jnp
from jax import lax
from jax.experimental import pallas as pl
from jax.experimental.pallas import tpu as pltpu
from jax.experimental.pallas import tpu_sc as plsc

B = 16384
D = 64
NC = 2
NS = 16
L = 16
NW = NC * NS
BPW = B // NW
CH = 128
CWID = 384
SLOTS = 544
IDPAD = B + (SLOTS - BPW) + L
TRASH = B

_mesh = plsc.VectorSubcoreMesh(core_axis_name="c", subcore_axis_name="s")
_params = pltpu.CompilerParams(use_tc_tiling_on_sc=True, needs_layout_passes=False)


def _make_extract(cmaxstart, cwid, nbuf):
    @functools.partial(
        pl.kernel,
        mesh=_mesh,
        out_type=jax.ShapeDtypeStruct((B + 8, 128), jnp.float32),
        scratch_types=[
            pltpu.VMEM((nbuf * 64, cwid), jnp.float32),
            pltpu.VMEM((SLOTS + L,), jnp.int32),
            pltpu.VMEM((4, CH), jnp.int32),
            pltpu.VMEM((SLOTS, 128), jnp.float32),
            pltpu.SemaphoreType.DMA,
            pltpu.SemaphoreType.DMA,
            pltpu.SemaphoreType.DMA,
        ],
        compiler_params=_params,
    )
    def _extract(ids_hbm, pos_hbm, tab_hbm, emb_hbm,
                 chunk, idv, psc, embv, sem, csem0, csem1):
        wid = lax.axis_index("s") * NC + lax.axis_index("c")
        base = wid * BPW
        pltpu.sync_copy(ids_hbm.at[pl.ds(base, SLOTS + L)], idv)
        for j in range(4):
            pltpu.sync_copy(pos_hbm.at[pl.ds(base + j * CH, CH)], psc.at[j])

        d16 = lax.broadcasted_iota(jnp.int32, (L,), 0)

        def chunk_start(r):
            u0 = idv[pl.ds(r, L)][0]
            return jnp.minimum((u0 // cwid) * cwid, jnp.int32(cmaxstart))

        def fire(q, cstart):
            csem = csem0 if q == 0 else csem1
            pltpu.async_copy(
                tab_hbm.at[:, pl.ds(pl.multiple_of(cstart, 128), cwid)],
                chunk.at[pl.ds(q * 64, 64)], csem)

        def drain(q):
            csem = csem0 if q == 0 else csem1
            pltpu.make_async_copy(
                tab_hbm.at[:, pl.ds(0, cwid)],
                chunk.at[pl.ds(q * 64, 64)], csem).wait()

        def extract(rowbase, r, uv0, cstart):
            for j in range(L):
                uj = uv0[j]
                cc = jnp.minimum(jnp.maximum(uj - cstart, 0), cwid - 1)
                ccv = jnp.zeros((L,), jnp.int32) + cc
                for k in range(4):
                    g = plsc.load_gather(chunk, [rowbase + d16 + k * L, ccv])
                    embv[r + j, pl.ds(k * L, L)] = g

        cs0 = chunk_start(jnp.int32(0))
        fire(0, cs0)

        if nbuf == 1:
            drain(0)

            def body1(carry):
                r, cstart = carry
                uv0 = idv[pl.ds(r, L)]
                m = (uv0 < cstart + cwid).astype(jnp.int32)
                n = jnp.sum(m)
                csn = chunk_start(r + n)
                extract(0, r, uv0, cstart)

                @pl.when(csn != cstart)
                def _():
                    fire(0, csn)
                    drain(0)

                return r + n, csn

            lax.while_loop(lambda c: c[0] < BPW, body1, (jnp.int32(0), cs0))
        else:
            def body2(carry):
                r, p, cstart, fresh = carry
                uv0 = idv[pl.ds(r, L)]
                m = (uv0 < cstart + cwid).astype(jnp.int32)
                n = jnp.sum(m)
                csn = chunk_start(r + n)
                moved = csn != cstart

                @pl.when(moved & (p == 0))
                def _():
                    fire(1, csn)

                @pl.when(moved & (p == 1))
                def _():
                    fire(0, csn)

                @pl.when((fresh == 1) & (p == 0))
                def _():
                    drain(0)

                @pl.when((fresh == 1) & (p == 1))
                def _():
                    drain(1)

                extract(p * 64, r, uv0, cstart)
                pn = jnp.where(moved, 1 - p, p)
                return r + n, pn, csn, moved.astype(jnp.int32)

            rf, pf, _, ff = lax.while_loop(
                lambda c: c[0] < BPW, body2,
                (jnp.int32(0), jnp.int32(0), cs0, jnp.int32(1)))

            @pl.when((ff == 1) & (pf == 0))
            def _():
                drain(0)

            @pl.when((ff == 1) & (pf == 1))
            def _():
                drain(1)

        for j in range(4):
            pltpu.async_copy(embv.at[pl.ds(j * CH, CH)],
                             emb_hbm.at[psc.at[j]], sem)
        for j in range(4):
            pltpu.make_async_copy(emb_hbm.at[pl.ds(0, CH)],
                                  embv.at[pl.ds(0, CH)], sem).wait()

    return _extract


_extract_user = _make_extract(1000064 - CWID, CWID, 2)
_extract_book = _make_extract(100096 - 768, 768, 1)


@functools.partial(
    pl.kernel,
    mesh=_mesh,
    out_type=jax.ShapeDtypeStruct((B,), jnp.float32),
    scratch_types=[
        pltpu.VMEM((BPW // 2, 128), jnp.float32),
        pltpu.VMEM((BPW // 2, 128), jnp.float32),
        pltpu.VMEM((BPW,), jnp.float32),
    ],
    compiler_params=_params,
)
def _dot(uemb_hbm, bemb_hbm, out_hbm, urows, brows, outv):
    wid = lax.axis_index("s") * NC + lax.axis_index("c")
    lane = lax.broadcasted_iota(jnp.int32, (L,), 0)
    HB = BPW // 2

    for h in range(2):
        hb = h * HB
        pltpu.sync_copy(uemb_hbm.at[pl.ds(wid * BPW + hb, HB), :], urows)
        pltpu.sync_copy(bemb_hbm.at[pl.ds(wid * BPW + hb, HB), :], brows)

        def group(g, carry):
            r0 = g * L
            acc = jnp.zeros((L,), jnp.float32)
            for j in range(L):
                r = r0 + j
                s = jnp.zeros((L,), jnp.float32)
                for k in range(D // L):
                    s = s + (urows[r, pl.ds(k * L, L)]
                             * brows[r, pl.ds(k * L, L)])
                acc = jnp.where(lane == j, jnp.sum(s), acc)
            outv[pl.ds(hb + r0, L)] = acc
            return carry

        lax.fori_loop(0, HB // L, group, 0, unroll=False)

    pltpu.sync_copy(outv, out_hbm.at[pl.ds(wid * BPW, BPW)])


def _sorted_ids(ids):
    iota = lax.broadcasted_iota(jnp.int32, (B,), 0)
    ids_sorted, perm = lax.sort((ids, iota), num_keys=1)
    npad = IDPAD - B
    ids_pad = jnp.concatenate(
        [ids_sorted, jnp.full((npad,), jnp.int32(0x7FFFFFFF))])
    pos_pad = jnp.concatenate(
        [perm, jnp.full((npad,), jnp.int32(TRASH))])
    return ids_pad, pos_pad


@jax.jit
def kernel(user_ids, book_ids, user_table, book_table):
    uid = user_ids.reshape(B)
    bid = book_ids.reshape(B)
    up, upos = _sorted_ids(uid)
    bp, bpos = _sorted_ids(bid)
    uemb = _extract_user(up, upos, user_table.T)
    bemb = _extract_book(bp, bpos, book_table.T)
    out = _dot(uemb, bemb)
    return out.reshape(B, 1)
